# baseline (speedup 1.0000x reference)
import functools

import jax
import jax.numpy as jnp
from jax.experimental import pallas as pl
from jax.experimental.pallas import tpu as pltpu

_TEMP = 0.5
_K = 10


def _bitonic_clean_desc(lst):
    n = len(lst)
    if n == 1:
        return lst
    h = n // 2
    hi = [jnp.maximum(lst[i], lst[i + h]) for i in range(h)]
    lo = [jnp.minimum(lst[i], lst[i + h]) for i in range(h)]
    return _bitonic_clean_desc(hi) + _bitonic_clean_desc(lo)


def _sort_desc(lst):
    n = len(lst)
    if n == 1:
        return lst
    a = _sort_desc(lst[: n // 2])
    b = _sort_desc(lst[n // 2:])
    return _bitonic_clean_desc(a + b[::-1])


def _topk_merge(a, b):
    m = [jnp.maximum(a[i], b[15 - i]) for i in range(16)]
    return _bitonic_clean_desc(m)


def _norm_kernel(z_ref, zn_ref):
    z = z_ref[...]
    s = jnp.sum(z * z, axis=1, keepdims=True)
    zn_ref[...] = (z * jax.lax.rsqrt(s)).astype(jnp.bfloat16)


def _loss_kernel(zn_blk_ref, zn_all_ref, acc_ref, *, rows, n, nblocks):
    i = pl.program_id(0)
    zb = zn_blk_ref[...]
    za = zn_all_ref[...]
    logits = jax.lax.dot_general(
        zb, za, (((1,), (1,)), ((), ())),
        preferred_element_type=jnp.float32)
    e = jnp.exp(logits * (1.0 / _TEMP))
    col = jax.lax.broadcasted_iota(jnp.int32, (rows, n), 1)
    row = jax.lax.broadcasted_iota(jnp.int32, (rows, n), 0) + i * rows
    is_diag = col == row
    e_all = jnp.sum(jnp.where(is_diag, 0.0, e), axis=1, keepdims=True)

    slices = [e[:, g * 128:(g + 1) * 128] for g in range(64)]
    runs = [_sort_desc(slices[i * 16:(i + 1) * 16]) for i in range(4)]
    ab = _topk_merge(runs[0], runs[1])
    cd = _topk_merge(runs[2], runs[3])
    cand = _topk_merge(ab, cd)

    for s in (1, 2, 4, 8, 16, 32, 64):
        rolled = [pltpu.roll(c, s, 1) for c in cand]
        cand = _topk_merge(cand, rolled)

    esum = cand[1]
    for r in range(2, _K + 1):
        esum = esum + cand[r]
    e_sim = jnp.max(esum, axis=1, keepdims=True)

    part = jnp.sum(jnp.log(e_all) - jnp.log(e_sim), axis=0, keepdims=True)

    @pl.when(i == 0)
    def _():
        acc_ref[...] = jnp.zeros((1, 1), jnp.float32)

    acc_ref[...] += part

    @pl.when(i == nblocks - 1)
    def _():
        acc_ref[...] = acc_ref[...] / n


def kernel(z):
    n, d = z.shape
    rows = 256
    nblocks = n // rows

    zn = pl.pallas_call(
        _norm_kernel,
        grid=(8,),
        in_specs=[pl.BlockSpec((n // 8, d), lambda i: (i, 0))],
        out_specs=pl.BlockSpec((n // 8, d), lambda i: (i, 0)),
        out_shape=jax.ShapeDtypeStruct((n, d), jnp.bfloat16),
    )(z)

    body = functools.partial(_loss_kernel, rows=rows, n=n, nblocks=nblocks)
    loss = pl.pallas_call(
        body,
        grid=(nblocks,),
        in_specs=[
            pl.BlockSpec((rows, d), lambda i: (i, 0)),
            pl.BlockSpec((n, d), lambda i: (0, 0)),
        ],
        out_specs=pl.BlockSpec((1, 1), lambda i: (0, 0)),
        out_shape=jax.ShapeDtypeStruct((1, 1), jnp.float32),
    )(zn, zn)

    return jnp.reshape(loss, ())


# speedup vs baseline: 1.1681x; 1.1681x over previous
import functools

import jax
import jax.numpy as jnp
from jax.experimental import pallas as pl
from jax.experimental.pallas import tpu as pltpu

_TEMP = 0.5
_K = 10


def _bitonic_clean_desc(lst):
    n = len(lst)
    if n == 1:
        return lst
    h = n // 2
    hi = [jnp.maximum(lst[i], lst[i + h]) for i in range(h)]
    lo = [jnp.minimum(lst[i], lst[i + h]) for i in range(h)]
    return _bitonic_clean_desc(hi) + _bitonic_clean_desc(lo)


def _sort_desc(lst):
    n = len(lst)
    if n == 1:
        return lst
    a = _sort_desc(lst[: n // 2])
    b = _sort_desc(lst[n // 2:])
    return _bitonic_clean_desc(a + b[::-1])


def _topk_merge(a, b):
    m = [jnp.maximum(a[i], b[15 - i]) for i in range(16)]
    return _bitonic_clean_desc(m)


def _norm_kernel(z_ref, zn_ref):
    z = z_ref[...]
    s = jnp.sum(z * z, axis=1, keepdims=True)
    zn_ref[...] = (z * jax.lax.rsqrt(s)).astype(jnp.bfloat16)


def _loss_kernel(zn_blk_ref, zn_all_ref, acc_ref, *, rows, n, nblocks):
    i = pl.program_id(0)
    zb = zn_blk_ref[...]
    za = zn_all_ref[...]
    logits = jax.lax.dot_general(
        zb, za, (((1,), (1,)), ((), ())),
        preferred_element_type=jnp.float32)
    e = jnp.exp(logits * (1.0 / _TEMP))
    col = jax.lax.broadcasted_iota(jnp.int32, (rows, n), 1)
    row = jax.lax.broadcasted_iota(jnp.int32, (rows, n), 0) + i * rows
    is_diag = col == row
    e_all = jnp.sum(jnp.where(is_diag, 0.0, e), axis=1, keepdims=True)

    slices = [e[:, g * 128:(g + 1) * 128] for g in range(64)]
    runs = [_sort_desc(slices[i * 16:(i + 1) * 16]) for i in range(4)]
    ab = _topk_merge(runs[0], runs[1])
    cd = _topk_merge(runs[2], runs[3])
    cand = _topk_merge(ab, cd)
    work = jnp.concatenate(cand[:_K + 1], axis=1)

    need = jnp.full((rows, 1), _K + 1, jnp.int32)
    topsum = jnp.zeros((rows, 1), jnp.float32)
    maxv = None
    for t in range(_K + 1):
        m = jnp.max(work, axis=1, keepdims=True)
        if t == 0:
            maxv = m
        eqm = work == m
        c = jnp.sum(eqm.astype(jnp.int32), axis=1, keepdims=True)
        take = jnp.minimum(c, need).astype(jnp.float32)
        topsum = topsum + take * m
        need = need - take.astype(jnp.int32)
        if t < _K:
            work = jnp.where(eqm, -jnp.inf, work)
    e_sim = topsum - maxv

    part = jnp.sum(jnp.log(e_all) - jnp.log(e_sim), axis=0, keepdims=True)

    @pl.when(i == 0)
    def _():
        acc_ref[...] = jnp.zeros((1, 1), jnp.float32)

    acc_ref[...] += part

    @pl.when(i == nblocks - 1)
    def _():
        acc_ref[...] = acc_ref[...] / n


def kernel(z):
    n, d = z.shape
    rows = 256
    nblocks = n // rows

    zn = pl.pallas_call(
        _norm_kernel,
        grid=(8,),
        in_specs=[pl.BlockSpec((n // 8, d), lambda i: (i, 0))],
        out_specs=pl.BlockSpec((n // 8, d), lambda i: (i, 0)),
        out_shape=jax.ShapeDtypeStruct((n, d), jnp.bfloat16),
    )(z)

    body = functools.partial(_loss_kernel, rows=rows, n=n, nblocks=nblocks)
    loss = pl.pallas_call(
        body,
        grid=(nblocks,),
        in_specs=[
            pl.BlockSpec((rows, d), lambda i: (i, 0)),
            pl.BlockSpec((n, d), lambda i: (0, 0)),
        ],
        out_specs=pl.BlockSpec((1, 1), lambda i: (0, 0)),
        out_shape=jax.ShapeDtypeStruct((1, 1), jnp.float32),
    )(zn, zn)

    return jnp.reshape(loss, ())


# speedup vs baseline: 1.6177x; 1.3849x over previous
import functools

import jax
import jax.numpy as jnp
from jax.experimental import pallas as pl
from jax.experimental.pallas import tpu as pltpu

_TEMP = 0.5
_K = 10


def _bitonic_clean_desc(lst):
    n = len(lst)
    if n == 1:
        return lst
    h = n // 2
    hi = [jnp.maximum(lst[i], lst[i + h]) for i in range(h)]
    lo = [jnp.minimum(lst[i], lst[i + h]) for i in range(h)]
    return _bitonic_clean_desc(hi) + _bitonic_clean_desc(lo)


def _sort_desc(lst):
    n = len(lst)
    if n == 1:
        return lst
    a = _sort_desc(lst[: n // 2])
    b = _sort_desc(lst[n // 2:])
    return _bitonic_clean_desc(a + b[::-1])


def _topk_merge(a, b):
    m = [jnp.maximum(a[i], b[15 - i]) for i in range(16)]
    return _bitonic_clean_desc(m)


def _norm_kernel(z_ref, zn_ref):
    z = z_ref[...]
    s = jnp.sum(z * z, axis=1, keepdims=True)
    zn_ref[...] = (z * jax.lax.rsqrt(s)).astype(jnp.bfloat16)


def _loss_kernel(zn_blk_ref, zn_all_ref, acc_ref, *, rows, n, nblocks):
    i = pl.program_id(0)
    zb = zn_blk_ref[...]
    za = zn_all_ref[...]
    logits = jax.lax.dot_general(
        zb, za, (((1,), (1,)), ((), ())),
        preferred_element_type=jnp.float32)
    e = jnp.exp(logits * (1.0 / _TEMP))
    col = jax.lax.broadcasted_iota(jnp.int32, (rows, n), 1)
    row = jax.lax.broadcasted_iota(jnp.int32, (rows, n), 0) + i * rows
    is_diag = col == row
    e_all = jnp.sum(jnp.where(is_diag, 0.0, e), axis=1, keepdims=True)

    e16 = e.astype(jnp.bfloat16)
    slices = [e16[:, g * 128:(g + 1) * 128] for g in range(64)]
    runs = [_sort_desc(slices[i * 16:(i + 1) * 16]) for i in range(4)]
    ab = _topk_merge(runs[0], runs[1])
    cd = _topk_merge(runs[2], runs[3])
    cand = _topk_merge(ab, cd)

    for s in (1, 2, 4, 8, 16, 32, 64):
        rolled = [pltpu.roll(c, s, 1) for c in cand]
        cand = _topk_merge(cand, rolled)

    esum = cand[1].astype(jnp.float32)
    for r in range(2, _K + 1):
        esum = esum + cand[r].astype(jnp.float32)
    e_sim = jnp.max(esum, axis=1, keepdims=True)

    part = jnp.sum(jnp.log(e_all) - jnp.log(e_sim), axis=0, keepdims=True)

    @pl.when(i == 0)
    def _():
        acc_ref[...] = jnp.zeros((1, 1), jnp.float32)

    acc_ref[...] += part

    @pl.when(i == nblocks - 1)
    def _():
        acc_ref[...] = acc_ref[...] / n


def kernel(z):
    n, d = z.shape
    rows = 256
    nblocks = n // rows

    zn = pl.pallas_call(
        _norm_kernel,
        grid=(8,),
        in_specs=[pl.BlockSpec((n // 8, d), lambda i: (i, 0))],
        out_specs=pl.BlockSpec((n // 8, d), lambda i: (i, 0)),
        out_shape=jax.ShapeDtypeStruct((n, d), jnp.bfloat16),
    )(z)

    body = functools.partial(_loss_kernel, rows=rows, n=n, nblocks=nblocks)
    loss = pl.pallas_call(
        body,
        grid=(nblocks,),
        in_specs=[
            pl.BlockSpec((rows, d), lambda i: (i, 0)),
            pl.BlockSpec((n, d), lambda i: (0, 0)),
        ],
        out_specs=pl.BlockSpec((1, 1), lambda i: (0, 0)),
        out_shape=jax.ShapeDtypeStruct((1, 1), jnp.float32),
    )(zn, zn)

    return jnp.reshape(loss, ())


# speedup vs baseline: 1.7572x; 1.0862x over previous
import functools

import jax
import jax.numpy as jnp
from jax.experimental import pallas as pl
from jax.experimental.pallas import tpu as pltpu

_TEMP = 0.5
_K = 10


def _bitonic_clean_desc(lst):
    n = len(lst)
    if n == 1:
        return lst
    h = n // 2
    hi = [jnp.maximum(lst[i], lst[i + h]) for i in range(h)]
    lo = [jnp.minimum(lst[i], lst[i + h]) for i in range(h)]
    return _bitonic_clean_desc(hi) + _bitonic_clean_desc(lo)


def _sort_desc(lst):
    n = len(lst)
    if n == 1:
        return lst
    a = _sort_desc(lst[: n // 2])
    b = _sort_desc(lst[n // 2:])
    return _bitonic_clean_desc(a + b[::-1])


def _topk_merge(a, b):
    m = [jnp.maximum(a[i], b[15 - i]) for i in range(16)]
    return _bitonic_clean_desc(m)


def _norm_kernel(z_ref, zn_ref):
    z = z_ref[...]
    s = jnp.sum(z * z, axis=1, keepdims=True)
    zn_ref[...] = (z * jax.lax.rsqrt(s)).astype(jnp.bfloat16)


def _loss_kernel(zn_blk_ref, zn_all_ref, acc_ref, *, rows, n, nblocks):
    i = pl.program_id(0)
    zb = zn_blk_ref[...]
    za = zn_all_ref[...]
    logits = jax.lax.dot_general(
        zb, za, (((1,), (1,)), ((), ())),
        preferred_element_type=jnp.float32)
    e = jnp.exp(logits * (1.0 / _TEMP))
    zbf = zb.astype(jnp.float32)
    diag = jnp.exp((2.0 / _TEMP) * jnp.sum(zbf * zbf, axis=1, keepdims=True))
    e_all = jnp.sum(e, axis=1, keepdims=True) - diag

    e16 = e.astype(jnp.bfloat16)
    slices = [e16[:, g * 128:(g + 1) * 128] for g in range(64)]
    runs = [_sort_desc(slices[i * 16:(i + 1) * 16]) for i in range(4)]
    ab = _topk_merge(runs[0], runs[1])
    cd = _topk_merge(runs[2], runs[3])
    cand = _topk_merge(ab, cd)

    for s in (1, 2, 4, 8, 16, 32, 64):
        rolled = [pltpu.roll(c, s, 1) for c in cand]
        cand = _topk_merge(cand, rolled)

    esum = cand[1].astype(jnp.float32)
    for r in range(2, _K + 1):
        esum = esum + cand[r].astype(jnp.float32)
    e_sim = jnp.max(esum, axis=1, keepdims=True)

    part = jnp.sum(jnp.log(e_all) - jnp.log(e_sim), axis=0, keepdims=True)

    @pl.when(i == 0)
    def _():
        acc_ref[...] = jnp.zeros((1, 1), jnp.float32)

    acc_ref[...] += part

    @pl.when(i == nblocks - 1)
    def _():
        acc_ref[...] = acc_ref[...] / n


def kernel(z):
    n, d = z.shape
    rows = 256
    nblocks = n // rows

    zn = pl.pallas_call(
        _norm_kernel,
        grid=(8,),
        in_specs=[pl.BlockSpec((n // 8, d), lambda i: (i, 0))],
        out_specs=pl.BlockSpec((n // 8, d), lambda i: (i, 0)),
        out_shape=jax.ShapeDtypeStruct((n, d), jnp.bfloat16),
    )(z)

    body = functools.partial(_loss_kernel, rows=rows, n=n, nblocks=nblocks)
    loss = pl.pallas_call(
        body,
        grid=(nblocks,),
        in_specs=[
            pl.BlockSpec((rows, d), lambda i: (i, 0)),
            pl.BlockSpec((n, d), lambda i: (0, 0)),
        ],
        out_specs=pl.BlockSpec((1, 1), lambda i: (0, 0)),
        out_shape=jax.ShapeDtypeStruct((1, 1), jnp.float32),
    )(zn, zn)

    return jnp.reshape(loss, ())


# speedup vs baseline: 1.7685x; 1.0064x over previous
import functools

import jax
import jax.numpy as jnp
from jax.experimental import pallas as pl
from jax.experimental.pallas import tpu as pltpu

_TEMP = 0.5
_K = 10


def _bitonic_clean_desc(lst):
    n = len(lst)
    if n == 1:
        return lst
    h = n // 2
    hi = [jnp.maximum(lst[i], lst[i + h]) for i in range(h)]
    lo = [jnp.minimum(lst[i], lst[i + h]) for i in range(h)]
    return _bitonic_clean_desc(hi) + _bitonic_clean_desc(lo)


def _sort_desc(lst):
    n = len(lst)
    if n == 1:
        return lst
    a = _sort_desc(lst[: n // 2])
    b = _sort_desc(lst[n // 2:])
    return _bitonic_clean_desc(a + b[::-1])


def _topk_merge(a, b):
    m = [jnp.maximum(a[i], b[15 - i]) for i in range(16)]
    return _bitonic_clean_desc(m)


def _norm_kernel(z_ref, zn_ref):
    z = z_ref[...]
    s = jnp.sum(z * z, axis=1, keepdims=True)
    zn_ref[...] = (z * jax.lax.rsqrt(s)).astype(jnp.bfloat16)


def _loss_kernel(zn_blk_ref, zn_all_ref, acc_ref, *, rows, n, nblocks):
    i = pl.program_id(0)
    zb = zn_blk_ref[...]
    za = zn_all_ref[...]
    logits = jax.lax.dot_general(
        zb, za, (((1,), (1,)), ((), ())),
        preferred_element_type=jnp.float32)
    e = jnp.exp(logits * (1.0 / _TEMP))
    zbf = zb.astype(jnp.float32)
    diag = jnp.exp((1.0 / _TEMP) * jnp.sum(zbf * zbf, axis=1, keepdims=True))
    e_all = jnp.sum(e, axis=1, keepdims=True) - diag

    e16 = e.astype(jnp.bfloat16)
    slices = [e16[:, g * 128:(g + 1) * 128] for g in range(64)]
    runs = [_sort_desc(slices[i * 16:(i + 1) * 16]) for i in range(4)]
    ab = _topk_merge(runs[0], runs[1])
    cd = _topk_merge(runs[2], runs[3])
    cand = _topk_merge(ab, cd)

    for s in (1, 2, 4, 8, 16, 32, 64):
        rolled = [pltpu.roll(c, s, 1) for c in cand]
        cand = _topk_merge(cand, rolled)

    esum = cand[1].astype(jnp.float32)
    for r in range(2, _K + 1):
        esum = esum + cand[r].astype(jnp.float32)
    e_sim = jnp.max(esum, axis=1, keepdims=True)

    part = jnp.sum(jnp.log(e_all) - jnp.log(e_sim), axis=0, keepdims=True)

    @pl.when(i == 0)
    def _():
        acc_ref[...] = jnp.zeros((1, 1), jnp.float32)

    acc_ref[...] += part

    @pl.when(i == nblocks - 1)
    def _():
        acc_ref[...] = acc_ref[...] / n


def kernel(z):
    n, d = z.shape
    rows = 256
    nblocks = n // rows

    zn = pl.pallas_call(
        _norm_kernel,
        grid=(8,),
        in_specs=[pl.BlockSpec((n // 8, d), lambda i: (i, 0))],
        out_specs=pl.BlockSpec((n // 8, d), lambda i: (i, 0)),
        out_shape=jax.ShapeDtypeStruct((n, d), jnp.bfloat16),
    )(z)

    body = functools.partial(_loss_kernel, rows=rows, n=n, nblocks=nblocks)
    loss = pl.pallas_call(
        body,
        grid=(nblocks,),
        in_specs=[
            pl.BlockSpec((rows, d), lambda i: (i, 0)),
            pl.BlockSpec((n, d), lambda i: (0, 0)),
        ],
        out_specs=pl.BlockSpec((1, 1), lambda i: (0, 0)),
        out_shape=jax.ShapeDtypeStruct((1, 1), jnp.float32),
    )(zn, zn)

    return jnp.reshape(loss, ())


# speedup vs baseline: 1.8083x; 1.0225x over previous
import functools

import jax
import jax.numpy as jnp
from jax.experimental import pallas as pl
from jax.experimental.pallas import tpu as pltpu

_TEMP = 0.5
_K = 10


def _bitonic_clean_desc(lst):
    n = len(lst)
    if n == 1:
        return lst
    h = n // 2
    hi = [jnp.maximum(lst[i], lst[i + h]) for i in range(h)]
    lo = [jnp.minimum(lst[i], lst[i + h]) for i in range(h)]
    return _bitonic_clean_desc(hi) + _bitonic_clean_desc(lo)


def _sort_desc(lst):
    n = len(lst)
    if n == 1:
        return lst
    a = _sort_desc(lst[: n // 2])
    b = _sort_desc(lst[n // 2:])
    return _bitonic_clean_desc(a + b[::-1])


def _topk_merge(a, b):
    m = [jnp.maximum(a[i], b[15 - i]) for i in range(16)]
    return _bitonic_clean_desc(m)


def _loss_kernel(z_all_ref, acc_ref, zn_scr, *, rows, n, nblocks):
    i = pl.program_id(0)

    @pl.when(i == 0)
    def _():
        z = z_all_ref[...]
        s = jnp.sum(z * z, axis=1, keepdims=True)
        zn_scr[...] = (z * jax.lax.rsqrt(s)).astype(jnp.bfloat16)

    za = zn_scr[...]
    zb = zn_scr[pl.ds(i * rows, rows), :]
    logits = jax.lax.dot_general(
        zb, za, (((1,), (1,)), ((), ())),
        preferred_element_type=jnp.float32)
    e = jnp.exp(logits * (1.0 / _TEMP))
    zbf = zb.astype(jnp.float32)
    diag = jnp.exp((1.0 / _TEMP) * jnp.sum(zbf * zbf, axis=1, keepdims=True))
    e_all = jnp.sum(e, axis=1, keepdims=True) - diag

    e16 = e.astype(jnp.bfloat16)
    slices = [e16[:, g * 128:(g + 1) * 128] for g in range(64)]
    runs = [_sort_desc(slices[i * 16:(i + 1) * 16]) for i in range(4)]
    ab = _topk_merge(runs[0], runs[1])
    cd = _topk_merge(runs[2], runs[3])
    cand = _topk_merge(ab, cd)

    for s in (1, 2, 4, 8, 16, 32, 64):
        rolled = [pltpu.roll(c, s, 1) for c in cand]
        cand = _topk_merge(cand, rolled)

    esum = cand[1].astype(jnp.float32)
    for r in range(2, _K + 1):
        esum = esum + cand[r].astype(jnp.float32)
    e_sim = jnp.max(esum, axis=1, keepdims=True)

    part = jnp.sum(jnp.log(e_all) - jnp.log(e_sim), axis=0, keepdims=True)

    @pl.when(i == 0)
    def _():
        acc_ref[...] = jnp.zeros((1, 1), jnp.float32)

    acc_ref[...] += part

    @pl.when(i == nblocks - 1)
    def _():
        acc_ref[...] = acc_ref[...] / n


def kernel(z):
    n, d = z.shape
    rows = 256
    nblocks = n // rows

    body = functools.partial(_loss_kernel, rows=rows, n=n, nblocks=nblocks)
    loss = pl.pallas_call(
        body,
        grid=(nblocks,),
        in_specs=[
            pl.BlockSpec((n, d), lambda i: (0, 0)),
        ],
        out_specs=pl.BlockSpec((1, 1), lambda i: (0, 0)),
        out_shape=jax.ShapeDtypeStruct((1, 1), jnp.float32),
        scratch_shapes=[pltpu.VMEM((n, d), jnp.bfloat16)],
    )(z)

    return jnp.reshape(loss, ())


# speedup vs baseline: 1.8756x; 1.0372x over previous
import functools

import jax
import jax.numpy as jnp
from jax.experimental import pallas as pl
from jax.experimental.pallas import tpu as pltpu

_TEMP = 0.5
_K = 10


def _bitonic_clean_desc(lst):
    n = len(lst)
    if n == 1:
        return lst
    h = n // 2
    hi = [jnp.maximum(lst[i], lst[i + h]) for i in range(h)]
    lo = [jnp.minimum(lst[i], lst[i + h]) for i in range(h)]
    return _bitonic_clean_desc(hi) + _bitonic_clean_desc(lo)


def _sort_desc(lst):
    n = len(lst)
    if n == 1:
        return lst
    a = _sort_desc(lst[: n // 2])
    b = _sort_desc(lst[n // 2:])
    return _bitonic_clean_desc(a + b[::-1])


def _topk_merge(a, b):
    m = [jnp.maximum(a[i], b[15 - i]) for i in range(16)]
    return _bitonic_clean_desc(m)


def _loss_kernel(z_all_ref, acc_ref, zn_scr, *, rows, n, nblocks):
    i = pl.program_id(0)

    @pl.when(i == 0)
    def _():
        z = z_all_ref[...]
        s = jnp.sum(z * z, axis=1, keepdims=True) * _TEMP
        zn_scr[...] = (z * jax.lax.rsqrt(s)).astype(jnp.bfloat16)

    za = zn_scr[...]
    zb = zn_scr[pl.ds(i * rows, rows), :]
    logits = jax.lax.dot_general(
        zb, za, (((1,), (1,)), ((), ())),
        preferred_element_type=jnp.float32)
    e = jnp.exp(logits)
    zbf = zb.astype(jnp.float32)
    diag = jnp.exp(jnp.sum(zbf * zbf, axis=1, keepdims=True))
    e_all = jnp.sum(e, axis=1, keepdims=True) - diag

    e16 = e.astype(jnp.bfloat16)
    slices = [e16[:, g * 128:(g + 1) * 128] for g in range(64)]
    runs = [_sort_desc(slices[i * 16:(i + 1) * 16]) for i in range(4)]
    ab = _topk_merge(runs[0], runs[1])
    cd = _topk_merge(runs[2], runs[3])
    cand = _topk_merge(ab, cd)

    for s in (1, 2, 4, 8, 16, 32, 64):
        rolled = [pltpu.roll(c, s, 1) for c in cand]
        cand = _topk_merge(cand, rolled)

    esum = cand[1].astype(jnp.float32)
    for r in range(2, _K + 1):
        esum = esum + cand[r].astype(jnp.float32)
    e_sim = jnp.max(esum, axis=1, keepdims=True)

    part = jnp.sum(jnp.log(e_all) - jnp.log(e_sim), axis=0, keepdims=True)

    @pl.when(i == 0)
    def _():
        acc_ref[...] = jnp.zeros((1, 1), jnp.float32)

    acc_ref[...] += part

    @pl.when(i == nblocks - 1)
    def _():
        acc_ref[...] = acc_ref[...] / n


def kernel(z):
    n, d = z.shape
    rows = 256
    nblocks = n // rows

    body = functools.partial(_loss_kernel, rows=rows, n=n, nblocks=nblocks)
    loss = pl.pallas_call(
        body,
        grid=(nblocks,),
        in_specs=[
            pl.BlockSpec((n, d), lambda i: (0, 0)),
        ],
        out_specs=pl.BlockSpec((1, 1), lambda i: (0, 0)),
        out_shape=jax.ShapeDtypeStruct((1, 1), jnp.float32),
        scratch_shapes=[pltpu.VMEM((n, d), jnp.bfloat16)],
    )(z)

    return jnp.reshape(loss, ())


# speedup vs baseline: 1.9124x; 1.0196x over previous
import functools

import jax
import jax.numpy as jnp
from jax.experimental import pallas as pl
from jax.experimental.pallas import tpu as pltpu

_TEMP = 0.5
_K = 10


def _bitonic_clean_desc(lst):
    n = len(lst)
    if n == 1:
        return lst
    h = n // 2
    hi = [jnp.maximum(lst[i], lst[i + h]) for i in range(h)]
    lo = [jnp.minimum(lst[i], lst[i + h]) for i in range(h)]
    return _bitonic_clean_desc(hi) + _bitonic_clean_desc(lo)


def _sort_desc(lst):
    n = len(lst)
    if n == 1:
        return lst
    a = _sort_desc(lst[: n // 2])
    b = _sort_desc(lst[n // 2:])
    return _bitonic_clean_desc(a + b[::-1])


def _topk_merge(a, b):
    m = [jnp.maximum(a[i], b[15 - i]) for i in range(16)]
    return _bitonic_clean_desc(m)


def _loss_kernel(z_all_ref, acc_ref, zn_scr, *, rows, n, nblocks):
    i = pl.program_id(0)

    @pl.when(i == 0)
    def _():
        z = z_all_ref[...]
        s = jnp.sum(z * z, axis=1, keepdims=True) * _TEMP
        zn_scr[...] = (z * jax.lax.rsqrt(s)).astype(jnp.bfloat16)

    za = zn_scr[...]
    zb = zn_scr[pl.ds(i * rows, rows), :]
    logits = jax.lax.dot_general(
        zb, za, (((1,), (1,)), ((), ())),
        preferred_element_type=jnp.float32)
    e = jnp.exp(logits)
    zbf = zb.astype(jnp.float32)
    diag = jnp.exp(jnp.sum(zbf * zbf, axis=1, keepdims=True))
    e_all = jnp.sum(e, axis=1, keepdims=True) - diag

    e16 = e.astype(jnp.bfloat16)
    slices = [e16[:, g * 128:(g + 1) * 128] for g in range(64)]
    runs = [_sort_desc(slices[i * 16:(i + 1) * 16]) for i in range(4)]
    ab = _topk_merge(runs[0], runs[1])
    cd = _topk_merge(runs[2], runs[3])
    cand = _topk_merge(ab, cd)

    for s in (1, 2, 4, 8, 16, 32, 64):
        rolled = [pltpu.roll(c, s, 1) for c in cand]
        cand = _topk_merge(cand, rolled)

    esum = cand[1].astype(jnp.float32)
    for r in range(2, _K + 1):
        esum = esum + cand[r].astype(jnp.float32)
    e_sim = jnp.max(esum, axis=1, keepdims=True)

    part = jnp.sum(jnp.log(e_all) - jnp.log(e_sim), axis=0, keepdims=True)

    @pl.when(i == 0)
    def _():
        acc_ref[...] = jnp.zeros((1, 1), jnp.float32)

    acc_ref[...] += part

    @pl.when(i == nblocks - 1)
    def _():
        acc_ref[...] = acc_ref[...] / n


def kernel(z):
    n, d = z.shape
    rows = 512
    nblocks = n // rows

    body = functools.partial(_loss_kernel, rows=rows, n=n, nblocks=nblocks)
    loss = pl.pallas_call(
        body,
        grid=(nblocks,),
        in_specs=[
            pl.BlockSpec((n, d), lambda i: (0, 0)),
        ],
        out_specs=pl.BlockSpec((1, 1), lambda i: (0, 0)),
        out_shape=jax.ShapeDtypeStruct((1, 1), jnp.float32),
        scratch_shapes=[pltpu.VMEM((n, d), jnp.bfloat16)],
    )(z)

    return jnp.reshape(loss, ())


# speedup vs baseline: 1.9474x; 1.0183x over previous
import functools

import jax
import jax.numpy as jnp
from jax.experimental import pallas as pl
from jax.experimental.pallas import tpu as pltpu

_TEMP = 0.5
_K = 10


def _bitonic_clean_desc(lst):
    n = len(lst)
    if n == 1:
        return lst
    h = n // 2
    hi = [jnp.maximum(lst[i], lst[i + h]) for i in range(h)]
    lo = [jnp.minimum(lst[i], lst[i + h]) for i in range(h)]
    return _bitonic_clean_desc(hi) + _bitonic_clean_desc(lo)


def _sort_desc(lst):
    n = len(lst)
    if n == 1:
        return lst
    a = _sort_desc(lst[: n // 2])
    b = _sort_desc(lst[n // 2:])
    return _bitonic_clean_desc(a + b[::-1])


def _topk_merge(a, b):
    m = [jnp.maximum(a[i], b[15 - i]) for i in range(16)]
    return _bitonic_clean_desc(m)


def _loss_kernel(z_all_ref, acc_ref, zn_scr, *, rows, n, nblocks):
    i = pl.program_id(0)

    @pl.when(i == 0)
    def _():
        z = z_all_ref[...]
        s = jnp.sum(z * z, axis=1, keepdims=True) * _TEMP
        zn_scr[...] = (z * jax.lax.rsqrt(s)).astype(jnp.bfloat16)

    za = zn_scr[...]
    zb = zn_scr[pl.ds(i * rows, rows), :]
    logits = jax.lax.dot_general(
        zb, za, (((1,), (1,)), ((), ())),
        preferred_element_type=jnp.float32)
    e = jnp.exp(logits)
    zbf = zb.astype(jnp.float32)
    diag = jnp.exp(jnp.sum(zbf * zbf, axis=1, keepdims=True))
    e_all = jnp.sum(e, axis=1, keepdims=True) - diag

    e16 = e.astype(jnp.bfloat16)
    slices = [e16[:, g * 128:(g + 1) * 128] for g in range(64)]
    runs = [_sort_desc(slices[i * 16:(i + 1) * 16]) for i in range(4)]
    ab = _topk_merge(runs[0], runs[1])
    cd = _topk_merge(runs[2], runs[3])
    cand = _topk_merge(ab, cd)

    for s in (1, 2, 4, 8, 16, 32, 64):
        rolled = [pltpu.roll(c, s, 1) for c in cand]
        cand = _topk_merge(cand, rolled)

    esum = cand[1].astype(jnp.float32)
    for r in range(2, _K + 1):
        esum = esum + cand[r].astype(jnp.float32)
    e_sim = jnp.max(esum, axis=1, keepdims=True)

    part = jnp.sum(jnp.log(e_all) - jnp.log(e_sim), axis=0, keepdims=True)

    @pl.when(i == 0)
    def _():
        acc_ref[...] = jnp.zeros((1, 1), jnp.float32)

    acc_ref[...] += part

    @pl.when(i == nblocks - 1)
    def _():
        acc_ref[...] = acc_ref[...] / n


def kernel(z):
    n, d = z.shape
    rows = 1024
    nblocks = n // rows

    body = functools.partial(_loss_kernel, rows=rows, n=n, nblocks=nblocks)
    loss = pl.pallas_call(
        body,
        grid=(nblocks,),
        in_specs=[
            pl.BlockSpec((n, d), lambda i: (0, 0)),
        ],
        out_specs=pl.BlockSpec((1, 1), lambda i: (0, 0)),
        out_shape=jax.ShapeDtypeStruct((1, 1), jnp.float32),
        scratch_shapes=[pltpu.VMEM((n, d), jnp.bfloat16)],
    )(z)

    return jnp.reshape(loss, ())


# speedup vs baseline: 2.0194x; 1.0370x over previous
import functools

import jax
import jax.numpy as jnp
from jax.experimental import pallas as pl
from jax.experimental.pallas import tpu as pltpu

_TEMP = 0.5
_K = 10


def _bitonic_clean_desc(lst):
    n = len(lst)
    if n == 1:
        return lst
    h = n // 2
    hi = [jnp.maximum(lst[i], lst[i + h]) for i in range(h)]
    lo = [jnp.minimum(lst[i], lst[i + h]) for i in range(h)]
    return _bitonic_clean_desc(hi) + _bitonic_clean_desc(lo)


def _sort_desc(lst):
    n = len(lst)
    if n == 1:
        return lst
    a = _sort_desc(lst[: n // 2])
    b = _sort_desc(lst[n // 2:])
    return _bitonic_clean_desc(a + b[::-1])


def _topk_merge(a, b):
    m = [jnp.maximum(a[i], b[15 - i]) for i in range(16)]
    return _bitonic_clean_desc(m)


def _loss_kernel(z_all_ref, acc_ref, zn_scr, *, rows, n, nblocks):
    i = pl.program_id(0)

    @pl.when(i == 0)
    def _():
        z = z_all_ref[...]
        s = jnp.sum(z * z, axis=1, keepdims=True) * (_TEMP / 1.4426950408889634)
        zn_scr[...] = (z * jax.lax.rsqrt(s)).astype(jnp.bfloat16)

    za = zn_scr[...]
    zb = zn_scr[pl.ds(i * rows, rows), :]
    logits = jax.lax.dot_general(
        zb, za, (((1,), (1,)), ((), ())),
        preferred_element_type=jnp.float32)
    e = jnp.exp2(logits)
    zbf = zb.astype(jnp.float32)
    diag = jnp.exp2(jnp.sum(zbf * zbf, axis=1, keepdims=True))
    ones = jnp.ones((n, 1), jnp.float32)
    e_all = jax.lax.dot_general(
        e, ones, (((1,), (0,)), ((), ())),
        preferred_element_type=jnp.float32) - diag

    e16 = e.astype(jnp.bfloat16)
    slices = [e16[:, g * 128:(g + 1) * 128] for g in range(64)]
    runs = [_sort_desc(slices[i * 16:(i + 1) * 16]) for i in range(4)]
    ab = _topk_merge(runs[0], runs[1])
    cd = _topk_merge(runs[2], runs[3])
    cand = _topk_merge(ab, cd)

    for s in (1, 2, 4, 8, 16, 32, 64):
        rolled = [pltpu.roll(c, s, 1) for c in cand]
        cand = _topk_merge(cand, rolled)

    esum = cand[1].astype(jnp.float32)
    for r in range(2, _K + 1):
        esum = esum + cand[r].astype(jnp.float32)
    e_sim = jnp.max(esum, axis=1, keepdims=True)

    part = jnp.sum(jnp.log(e_all) - jnp.log(e_sim), axis=0, keepdims=True)

    @pl.when(i == 0)
    def _():
        acc_ref[...] = jnp.zeros((1, 1), jnp.float32)

    acc_ref[...] += part

    @pl.when(i == nblocks - 1)
    def _():
        acc_ref[...] = acc_ref[...] / n


def kernel(z):
    n, d = z.shape
    rows = 1024
    nblocks = n // rows

    body = functools.partial(_loss_kernel, rows=rows, n=n, nblocks=nblocks)
    loss = pl.pallas_call(
        body,
        grid=(nblocks,),
        in_specs=[
            pl.BlockSpec((n, d), lambda i: (0, 0)),
        ],
        out_specs=pl.BlockSpec((1, 1), lambda i: (0, 0)),
        out_shape=jax.ShapeDtypeStruct((1, 1), jnp.float32),
        scratch_shapes=[pltpu.VMEM((n, d), jnp.bfloat16)],
    )(z)

    return jnp.reshape(loss, ())


# speedup vs baseline: 2.1186x; 1.0491x over previous
import functools

import jax
import jax.numpy as jnp
from jax.experimental import pallas as pl
from jax.experimental.pallas import tpu as pltpu

_TEMP = 0.5
_K = 10


def _bitonic_clean_desc(lst):
    n = len(lst)
    if n == 1:
        return lst
    h = n // 2
    hi = [jnp.maximum(lst[i], lst[i + h]) for i in range(h)]
    lo = [jnp.minimum(lst[i], lst[i + h]) for i in range(h)]
    return _bitonic_clean_desc(hi) + _bitonic_clean_desc(lo)


def _oe_merge(a, b):
    n = len(a)
    if n == 1:
        return [jnp.maximum(a[0], b[0]), jnp.minimum(a[0], b[0])]
    even = _oe_merge(a[::2], b[::2])
    odd = _oe_merge(a[1::2], b[1::2])
    res = [even[0]]
    for i in range(n - 1):
        res.append(jnp.maximum(odd[i], even[i + 1]))
        res.append(jnp.minimum(odd[i], even[i + 1]))
    res.append(odd[n - 1])
    return res


def _sort_desc(lst):
    n = len(lst)
    if n == 1:
        return lst
    return _oe_merge(_sort_desc(lst[: n // 2]), _sort_desc(lst[n // 2:]))


def _topk_merge(a, b):
    m = [jnp.maximum(a[i], b[15 - i]) for i in range(16)]
    return _bitonic_clean_desc(m)


def _loss_kernel(z_all_ref, acc_ref, zn_scr, *, rows, n, nblocks):
    i = pl.program_id(0)

    @pl.when(i == 0)
    def _():
        z = z_all_ref[...]
        s = jnp.sum(z * z, axis=1, keepdims=True) * (_TEMP / 1.4426950408889634)
        zn_scr[...] = (z * jax.lax.rsqrt(s)).astype(jnp.bfloat16)

    za = zn_scr[...]
    zb = zn_scr[pl.ds(i * rows, rows), :]
    logits = jax.lax.dot_general(
        zb, za, (((1,), (1,)), ((), ())),
        preferred_element_type=jnp.float32)
    e = jnp.exp2(logits)
    zbf = zb.astype(jnp.float32)
    diag = jnp.exp2(jnp.sum(zbf * zbf, axis=1, keepdims=True))
    ones = jnp.ones((n, 1), jnp.float32)
    e_all = jax.lax.dot_general(
        e, ones, (((1,), (0,)), ((), ())),
        preferred_element_type=jnp.float32) - diag

    e16 = e.astype(jnp.bfloat16)
    slices = [e16[:, g * 128:(g + 1) * 128] for g in range(64)]
    runs = [_sort_desc(slices[i * 16:(i + 1) * 16]) for i in range(4)]
    ab = _topk_merge(runs[0], runs[1])
    cd = _topk_merge(runs[2], runs[3])
    cand = _topk_merge(ab, cd)

    for s in (1, 2, 4, 8, 16, 32, 64):
        rolled = [pltpu.roll(c, s, 1) for c in cand]
        cand = _topk_merge(cand, rolled)

    esum = cand[1].astype(jnp.float32)
    for r in range(2, _K + 1):
        esum = esum + cand[r].astype(jnp.float32)
    e_sim = jnp.max(esum, axis=1, keepdims=True)

    part = jnp.sum(jnp.log(e_all) - jnp.log(e_sim), axis=0, keepdims=True)

    @pl.when(i == 0)
    def _():
        acc_ref[...] = jnp.zeros((1, 1), jnp.float32)

    acc_ref[...] += part

    @pl.when(i == nblocks - 1)
    def _():
        acc_ref[...] = acc_ref[...] / n


def kernel(z):
    n, d = z.shape
    rows = 1024
    nblocks = n // rows

    body = functools.partial(_loss_kernel, rows=rows, n=n, nblocks=nblocks)
    loss = pl.pallas_call(
        body,
        grid=(nblocks,),
        in_specs=[
            pl.BlockSpec((n, d), lambda i: (0, 0)),
        ],
        out_specs=pl.BlockSpec((1, 1), lambda i: (0, 0)),
        out_shape=jax.ShapeDtypeStruct((1, 1), jnp.float32),
        scratch_shapes=[pltpu.VMEM((n, d), jnp.bfloat16)],
    )(z)

    return jnp.reshape(loss, ())
